# baseline (speedup 1.0000x reference)
import jax
import jax.numpy as jnp
from jax.experimental import pallas as pl
from jax.experimental.pallas import tpu as pltpu

_BP = 2048


def _lvq_block(x_ref, p_ref, out_ref):
    x = x_ref[...]
    p = p_ref[...]
    x2 = jnp.sum(x * x, axis=1, keepdims=True)
    p2 = jnp.sum(p * p, axis=1)
    dot = jax.lax.dot_general(
        x, p, (((1,), (1,)), ((), ())), preferred_element_type=jnp.float32
    )
    sq = x2 + p2[None, :] - 2.0 * dot
    out_ref[...] = -jnp.sqrt(jnp.maximum(sq, 1e-12))


def kernel(x, prototypes):
    B, D = x.shape
    P = prototypes.shape[0]
    return pl.pallas_call(
        _lvq_block,
        grid=(pl.cdiv(P, _BP),),
        in_specs=[
            pl.BlockSpec((B, D), lambda i: (0, 0)),
            pl.BlockSpec((_BP, D), lambda i: (i, 0)),
        ],
        out_specs=pl.BlockSpec((B, _BP), lambda i: (0, i)),
        out_shape=jax.ShapeDtypeStruct((B, P), jnp.float32),
        compiler_params=pltpu.CompilerParams(
            dimension_semantics=("arbitrary",),
        ),
    )(x, prototypes)


# speedup vs baseline: 1.1004x; 1.1004x over previous
import jax
import jax.numpy as jnp
from jax.experimental import pallas as pl
from jax.experimental.pallas import tpu as pltpu

_BP = 2048


def _lvq_block(x_ref, p_ref, out_ref):
    x = x_ref[...]
    p = p_ref[...]
    x2 = jnp.sum(x * x, axis=1, keepdims=True)
    p2 = jnp.sum(p * p, axis=1, keepdims=True)
    ones_x = jnp.ones_like(x2)
    ones_p = jnp.ones_like(p2)
    x_aug = jnp.concatenate([-2.0 * x, x2, ones_x], axis=1)
    p_aug = jnp.concatenate([p, ones_p, p2], axis=1)
    sq = jax.lax.dot_general(
        x_aug, p_aug, (((1,), (1,)), ((), ())),
        preferred_element_type=jnp.float32,
    )
    m = jnp.maximum(sq, 1e-12)
    out_ref[...] = -(m * jax.lax.rsqrt(m))


def kernel(x, prototypes):
    B, D = x.shape
    P = prototypes.shape[0]
    return pl.pallas_call(
        _lvq_block,
        grid=(pl.cdiv(P, _BP),),
        in_specs=[
            pl.BlockSpec((B, D), lambda i: (0, 0)),
            pl.BlockSpec((_BP, D), lambda i: (i, 0)),
        ],
        out_specs=pl.BlockSpec((B, _BP), lambda i: (0, i)),
        out_shape=jax.ShapeDtypeStruct((B, P), jnp.float32),
        compiler_params=pltpu.CompilerParams(
            dimension_semantics=("arbitrary",),
        ),
    )(x, prototypes)


# speedup vs baseline: 1.1062x; 1.0052x over previous
import jax
import jax.numpy as jnp
from jax.experimental import pallas as pl
from jax.experimental.pallas import tpu as pltpu

_BP = 4096


def _lvq_block(x_ref, p_ref, out_ref):
    x = x_ref[...]
    p = p_ref[...]
    x2 = jnp.sum(x * x, axis=1, keepdims=True)
    p2 = jnp.sum(p * p, axis=1, keepdims=True)
    ones_x = jnp.ones_like(x2)
    ones_p = jnp.ones_like(p2)
    x_aug = jnp.concatenate([-2.0 * x, x2, ones_x], axis=1)
    p_aug = jnp.concatenate([p, ones_p, p2], axis=1)
    sq = jax.lax.dot_general(
        x_aug, p_aug, (((1,), (1,)), ((), ())),
        preferred_element_type=jnp.float32,
    )
    m = jnp.maximum(sq, 1e-12)
    out_ref[...] = -(m * jax.lax.rsqrt(m))


def kernel(x, prototypes):
    B, D = x.shape
    P = prototypes.shape[0]
    return pl.pallas_call(
        _lvq_block,
        grid=(pl.cdiv(P, _BP),),
        in_specs=[
            pl.BlockSpec((B, D), lambda i: (0, 0)),
            pl.BlockSpec((_BP, D), lambda i: (i, 0)),
        ],
        out_specs=pl.BlockSpec((B, _BP), lambda i: (0, i)),
        out_shape=jax.ShapeDtypeStruct((B, P), jnp.float32),
        compiler_params=pltpu.CompilerParams(
            dimension_semantics=("arbitrary",),
        ),
    )(x, prototypes)
